# baseline (device time: 16134 ns/iter reference)
import jax
import jax.numpy as jnp
from jax import lax
from jax.experimental import pallas as pl
from jax.experimental.pallas import tpu as pltpu

T = 256
D = 512
V_SHARD = 4096
NC = 8
VC = V_SHARD // NC


def kernel(x, W, labels):
    labels2d = labels.reshape(T, 1)

    def body(
        x_ref, w_ref, lab_ref, out_ref, acc_ref, send_buf, recv_buf, send_sem, recv_sem
    ):
        j = pl.program_id(0)
        my_x = lax.axis_index("x")
        my_y = lax.axis_index("y")
        my_z = lax.axis_index("z")
        nbr = (my_x, my_y, 1 - my_z)

        barrier_sem = pltpu.get_barrier_semaphore()

        @pl.when(j == 0)
        def _():
            pl.semaphore_signal(
                barrier_sem, inc=1, device_id=nbr, device_id_type=pl.DeviceIdType.MESH
            )
            acc_ref[:, :] = jnp.zeros((T, 2), jnp.float32)

        logits = jnp.dot(
            x_ref[:, :].astype(jnp.bfloat16),
            w_ref[:, :].astype(jnp.bfloat16),
            preferred_element_type=jnp.float32,
        )
        s = jnp.sum(jnp.exp(logits), axis=1, keepdims=True)
        cols = lax.broadcasted_iota(jnp.int32, (T, VC), 1) + j * VC
        local_label = lab_ref[:, :] - my_z * V_SHARD
        t = jnp.sum(
            jnp.where(cols == local_label, logits, 0.0), axis=1, keepdims=True
        )
        acc_ref[:, :] += jnp.concatenate([s, t], axis=1)

        @pl.when(j == NC - 1)
        def _():
            send_buf[:, :] = acc_ref[:, :]
            pl.semaphore_wait(barrier_sem, 1)
            rdma = pltpu.make_async_remote_copy(
                src_ref=send_buf,
                dst_ref=recv_buf,
                send_sem=send_sem,
                recv_sem=recv_sem,
                device_id=nbr,
                device_id_type=pl.DeviceIdType.MESH,
            )
            rdma.start()
            rdma.wait()
            sg = acc_ref[:, 0:1] + recv_buf[:, 0:1]
            tg = acc_ref[:, 1:2] + recv_buf[:, 1:2]
            out_ref[:, :] = jnp.log(sg) - tg

    out = pl.pallas_call(
        body,
        grid=(NC,),
        out_shape=jax.ShapeDtypeStruct((T, 1), jnp.float32),
        in_specs=[
            pl.BlockSpec((T, D), lambda j: (0, 0), memory_space=pltpu.VMEM),
            pl.BlockSpec((D, VC), lambda j: (0, j), memory_space=pltpu.VMEM),
            pl.BlockSpec((T, 1), lambda j: (0, 0), memory_space=pltpu.VMEM),
        ],
        out_specs=pl.BlockSpec((T, 1), lambda j: (0, 0), memory_space=pltpu.VMEM),
        scratch_shapes=[
            pltpu.VMEM((T, 2), jnp.float32),
            pltpu.VMEM((T, 2), jnp.float32),
            pltpu.VMEM((T, 2), jnp.float32),
            pltpu.SemaphoreType.DMA,
            pltpu.SemaphoreType.DMA,
        ],
        compiler_params=pltpu.CompilerParams(collective_id=0),
    )(x, W, labels2d)
    return out.reshape(T)


# device time: 16076 ns/iter; 1.0036x vs baseline; 1.0036x over previous
import jax
import jax.numpy as jnp
from jax import lax
from jax.experimental import pallas as pl
from jax.experimental.pallas import tpu as pltpu

T = 256
D = 512
V_SHARD = 4096
NC = 8
VC = V_SHARD // NC


def kernel(x, W, labels):
    labels2d = labels.reshape(T, 1)

    def body(
        x_ref, w_ref, lab_ref, out_ref, acc_ref, send_buf, recv_buf, send_sem, recv_sem
    ):
        j = pl.program_id(0)
        my_x = lax.axis_index("x")
        my_y = lax.axis_index("y")
        my_z = lax.axis_index("z")
        nbr = (my_x, my_y, 1 - my_z)

        barrier_sem = pltpu.get_barrier_semaphore()

        @pl.when(j == 0)
        def _():
            pl.semaphore_signal(
                barrier_sem, inc=1, device_id=nbr, device_id_type=pl.DeviceIdType.MESH
            )
            acc_ref[:, :] = jnp.zeros((T, 2), jnp.float32)

        logits = jnp.dot(
            x_ref[:, :].astype(jnp.bfloat16),
            w_ref[:, :].astype(jnp.bfloat16),
            preferred_element_type=jnp.float32,
        )
        s = jnp.sum(jnp.exp(logits), axis=1, keepdims=True)
        cols = lax.broadcasted_iota(jnp.int32, (T, VC), 1) + j * VC
        local_label = lab_ref[:, :] - my_z * V_SHARD
        t = jnp.sum(
            jnp.where(cols == local_label, logits, 0.0), axis=1, keepdims=True
        )
        acc_ref[:, :] += jnp.concatenate([s, t], axis=1)

        @pl.when(j == NC - 1)
        def _():
            send_buf[:, :] = acc_ref[:, :]
            pl.semaphore_wait(barrier_sem, 1)
            rdma = pltpu.make_async_remote_copy(
                src_ref=send_buf,
                dst_ref=recv_buf,
                send_sem=send_sem,
                recv_sem=recv_sem,
                device_id=nbr,
                device_id_type=pl.DeviceIdType.MESH,
            )
            rdma.start()
            rdma.wait()
            sg = acc_ref[:, 0:1] + recv_buf[:, 0:1]
            tg = acc_ref[:, 1:2] + recv_buf[:, 1:2]
            out_ref[:] = (jnp.log(sg) - tg)[:, 0]

    out = pl.pallas_call(
        body,
        grid=(NC,),
        out_shape=jax.ShapeDtypeStruct((T,), jnp.float32),
        in_specs=[
            pl.BlockSpec((T, D), lambda j: (0, 0)),
            pl.BlockSpec((D, VC), lambda j: (0, j)),
            pl.BlockSpec((T, 1), lambda j: (0, 0)),
        ],
        out_specs=pl.BlockSpec((T,), lambda j: (0,)),
        scratch_shapes=[
            pltpu.VMEM((T, 2), jnp.float32),
            pltpu.VMEM((T, 2), jnp.float32),
            pltpu.VMEM((T, 2), jnp.float32),
            pltpu.SemaphoreType.DMA,
            pltpu.SemaphoreType.DMA,
        ],
        compiler_params=pltpu.CompilerParams(collective_id=0),
    )(x, W, labels2d)
    return out
